# baseline (speedup 1.0000x reference)
import functools

import jax
import jax.numpy as jnp
from jax import lax
from jax.experimental import pallas as pl
from jax.experimental.pallas import tpu as pltpu
from jax.experimental.pallas import tpu_sc as plsc

_B, _P, _C, _K = 4, 4096, 64, 16
_R = 512
_R2 = 256
_CH = 128
_HIGHEST = lax.Precision.HIGHEST
_DEFAULT = lax.Precision.DEFAULT


def _leaky(x):
    return jnp.where(x >= 0, x, 0.2 * x)


def _sim_topk_body(nodes_full_ref, nodes_blk_ref, wd_ref, wb_ref, b1_ref,
                   knn_ref, a_ref, nb_ref, nn_ref):
    i = pl.program_id(0)
    x_blk = nodes_blk_ref[...]

    @pl.when(i == 0)
    def _():
        x_full = nodes_full_ref[...]
        ssq = jnp.sum(x_full * x_full, axis=1, keepdims=True)
        nn_ref[...] = x_full / jnp.maximum(jnp.sqrt(ssq), 1e-12)

    nn_full = nn_ref[...]
    nn_blk = nn_ref[pl.ds(i * _R, _R), :]
    sim = lax.dot_general(nn_blk, nn_full, (((1,), (1,)), ((), ())),
                          preferred_element_type=jnp.float32,
                          precision=_DEFAULT)
    col = lax.broadcasted_iota(jnp.int32, (1, _P), 1)
    row = lax.broadcasted_iota(jnp.int32, (_R, 1), 0) + i * _R
    neg_inf = jnp.float32(-jnp.inf)
    run = jnp.where(col == row, neg_inf, sim)
    idxs = []
    for t in range(_K):
        if t > 0:
            run = jnp.where(col == idxs[-1], neg_inf, run)
        idxs.append(jnp.argmax(run, axis=1).astype(jnp.int32)[:, None])
    knn_ref[...] = jnp.concatenate(idxs, axis=1)
    a_ref[...] = lax.dot_general(x_blk, wd_ref[...], (((1,), (0,)), ((), ())),
                                 preferred_element_type=jnp.float32,
                                 precision=_DEFAULT) + b1_ref[...]
    nbv = lax.dot_general(x_blk, wb_ref[...], (((1,), (0,)), ((), ())),
                          preferred_element_type=jnp.float32,
                          precision=_DEFAULT)
    nb_ref[...] = jnp.concatenate([nbv, jnp.zeros((_R, _C), jnp.float32)],
                                  axis=1)


def _sim_topk(nodes_b, wd, wb, b1_2d):
    return pl.pallas_call(
        _sim_topk_body,
        grid=(_P // _R,),
        in_specs=[
            pl.BlockSpec((_P, _C), lambda i: (0, 0)),
            pl.BlockSpec((_R, _C), lambda i: (i, 0)),
            pl.BlockSpec((_C, _C), lambda i: (0, 0)),
            pl.BlockSpec((_C, _C), lambda i: (0, 0)),
            pl.BlockSpec((1, _C), lambda i: (0, 0)),
        ],
        out_specs=[
            pl.BlockSpec((_R, _K), lambda i: (i, 0)),
            pl.BlockSpec((_R, _C), lambda i: (i, 0)),
            pl.BlockSpec((_R, 2 * _C), lambda i: (i, 0)),
        ],
        out_shape=[
            jax.ShapeDtypeStruct((_P, _K), jnp.int32),
            jax.ShapeDtypeStruct((_P, _C), jnp.float32),
            jax.ShapeDtypeStruct((_P, 2 * _C), jnp.float32),
        ],
        scratch_shapes=[pltpu.VMEM((_P, _C), jnp.float32)],
    )(nodes_b, nodes_b, wd, wb, b1_2d)


def _sc_gather(table, idx_flat):
    n = idx_flat.shape[0]
    info = plsc.get_sparse_core_info()
    nw = info.num_cores * info.num_subcores
    per_w = n // nw
    n_ch = per_w // _CH
    mesh = plsc.VectorSubcoreMesh(core_axis_name="c", subcore_axis_name="s")

    @functools.partial(
        pl.kernel, mesh=mesh,
        out_type=jax.ShapeDtypeStruct((n, 2 * _C), jnp.float32),
        scratch_types=[
            pltpu.VMEM((_CH,), jnp.int32),
            pltpu.VMEM((_CH, 2 * _C), jnp.float32),
            pltpu.SemaphoreType.DMA,
        ],
    )
    def k(table_hbm, idx_hbm, out_hbm, idx_v, rows_v, sem):
        wid = lax.axis_index("s") * info.num_cores + lax.axis_index("c")
        base0 = wid * per_w

        @pl.loop(0, n_ch)
        def _(j):
            base = base0 + j * _CH
            pltpu.sync_copy(idx_hbm.at[pl.ds(base, _CH)], idx_v)
            pltpu.async_copy(table_hbm.at[idx_v], rows_v, sem).wait()
            pltpu.sync_copy(rows_v, out_hbm.at[pl.ds(base, _CH)])

    return k(table, idx_flat)


def _mlp_body(a_ref, g_ref, w2t_ref, b2_ref, gamma_ref, beta_ref, o_ref):
    a = a_ref[...]
    acc = jnp.zeros((_R2, _C), jnp.float32)
    for t in range(_K):
        h1 = _leaky(a + g_ref[t][:, :_C])
        z = lax.dot_general(h1, w2t_ref[...], (((1,), (0,)), ((), ())),
                            preferred_element_type=jnp.float32,
                            precision=_HIGHEST) + b2_ref[...]
        acc = acc + _leaky(z)
    agg = acc * (1.0 / _K)
    mu = jnp.mean(agg, axis=1, keepdims=True)
    d = agg - mu
    var = jnp.mean(d * d, axis=1, keepdims=True)
    o_ref[...] = (d / jnp.sqrt(var + 1e-5)) * gamma_ref[...] + beta_ref[...]


def _mlp(a2, g, w2t, b2_2d, gamma_2d, beta_2d):
    npts = a2.shape[0]
    return pl.pallas_call(
        _mlp_body,
        grid=(npts // _R2,),
        in_specs=[
            pl.BlockSpec((_R2, _C), lambda i: (i, 0)),
            pl.BlockSpec((_K, _R2, 2 * _C), lambda i: (0, i, 0)),
            pl.BlockSpec((_C, _C), lambda i: (0, 0)),
            pl.BlockSpec((1, _C), lambda i: (0, 0)),
            pl.BlockSpec((1, _C), lambda i: (0, 0)),
            pl.BlockSpec((1, _C), lambda i: (0, 0)),
        ],
        out_specs=pl.BlockSpec((_R2, _C), lambda i: (i, 0)),
        out_shape=jax.ShapeDtypeStruct((npts, _C), jnp.float32),
    )(a2, g, w2t, b2_2d, gamma_2d, beta_2d)


def kernel(nodes, w1, b1, w2, b2, gamma, beta):
    b_, p_, c_ = nodes.shape
    w1t = w1.T
    wa = w1t[:c_]
    wb = w1t[c_:]
    wd = wa - wb
    b1_2d = b1.reshape(1, c_)
    w2t = w2.T
    b2_2d = b2.reshape(1, c_)
    gamma_2d = gamma.reshape(1, c_)
    beta_2d = beta.reshape(1, c_)
    outs = []
    for b in range(b_):
        knn, a, nb = _sim_topk(nodes[b], wd, wb, b1_2d)
        idx_flat = knn.T.reshape(-1)
        g = _sc_gather(nb, idx_flat)
        g = g.reshape(_K, p_, 2 * c_)
        outs.append(_mlp(a, g, w2t, b2_2d, gamma_2d, beta_2d))
    return jnp.stack(outs, axis=0)


# speedup vs baseline: 1.0466x; 1.0466x over previous
import functools

import jax
import jax.numpy as jnp
from jax import lax
from jax.experimental import pallas as pl
from jax.experimental.pallas import tpu as pltpu
from jax.experimental.pallas import tpu_sc as plsc

_B, _P, _C, _K = 4, 4096, 64, 16
_R = 512
_R2 = 256
_CH = 128
_HIGHEST = lax.Precision.HIGHEST
_DEFAULT = lax.Precision.DEFAULT


def _leaky(x):
    return jnp.where(x >= 0, x, 0.2 * x)


def _sim_topk_body(nodes_full_ref, nodes_blk_ref, wd_ref, wb_ref, b1_ref,
                   knn_ref, a_ref, nb_ref, nn_ref):
    i = pl.program_id(0)
    x_blk = nodes_blk_ref[...]

    @pl.when(i == 0)
    def _():
        x_full = nodes_full_ref[...]
        ssq = jnp.sum(x_full * x_full, axis=1, keepdims=True)
        nn_ref[...] = x_full / jnp.maximum(jnp.sqrt(ssq), 1e-12)

    nn_full = nn_ref[...]
    nn_blk = nn_ref[pl.ds(i * _R, _R), :]
    sim = lax.dot_general(nn_blk, nn_full, (((1,), (1,)), ((), ())),
                          preferred_element_type=jnp.float32,
                          precision=_DEFAULT)
    col = lax.broadcasted_iota(jnp.int32, (1, _P), 1)
    row = lax.broadcasted_iota(jnp.int32, (_R, 1), 0) + i * _R
    neg_inf = jnp.float32(-jnp.inf)
    run = jnp.where(col == row, neg_inf, sim)
    idxs = []
    for t in range(_K):
        if t > 0:
            run = jnp.where(col == idxs[-1], neg_inf, run)
        idxs.append(jnp.argmax(run, axis=1).astype(jnp.int32)[:, None])
    knn_ref[...] = jnp.concatenate(idxs, axis=1)
    a_ref[...] = lax.dot_general(x_blk, wd_ref[...], (((1,), (0,)), ((), ())),
                                 preferred_element_type=jnp.float32,
                                 precision=_DEFAULT) + b1_ref[...]
    nbv = lax.dot_general(x_blk, wb_ref[...], (((1,), (0,)), ((), ())),
                          preferred_element_type=jnp.float32,
                          precision=_DEFAULT)
    nb_ref[...] = jnp.concatenate([nbv, jnp.zeros((_R, _C), jnp.float32)],
                                  axis=1)


def _sim_topk(nodes_b, wd, wb, b1_2d):
    return pl.pallas_call(
        _sim_topk_body,
        grid=(_P // _R,),
        in_specs=[
            pl.BlockSpec((_P, _C), lambda i: (0, 0)),
            pl.BlockSpec((_R, _C), lambda i: (i, 0)),
            pl.BlockSpec((_C, _C), lambda i: (0, 0)),
            pl.BlockSpec((_C, _C), lambda i: (0, 0)),
            pl.BlockSpec((1, _C), lambda i: (0, 0)),
        ],
        out_specs=[
            pl.BlockSpec((_R, _K), lambda i: (i, 0)),
            pl.BlockSpec((_R, _C), lambda i: (i, 0)),
            pl.BlockSpec((_R, 2 * _C), lambda i: (i, 0)),
        ],
        out_shape=[
            jax.ShapeDtypeStruct((_P, _K), jnp.int32),
            jax.ShapeDtypeStruct((_P, _C), jnp.float32),
            jax.ShapeDtypeStruct((_P, 2 * _C), jnp.float32),
        ],
        scratch_shapes=[pltpu.VMEM((_P, _C), jnp.float32)],
    )(nodes_b, nodes_b, wd, wb, b1_2d)


def _sc_gather(table, idx_flat):
    n = idx_flat.shape[0]
    info = plsc.get_sparse_core_info()
    nw = info.num_cores * info.num_subcores
    per_w = n // nw
    n_ch = per_w // _CH
    mesh = plsc.VectorSubcoreMesh(core_axis_name="c", subcore_axis_name="s")

    @functools.partial(
        pl.kernel, mesh=mesh,
        out_type=jax.ShapeDtypeStruct((n, 2 * _C), jnp.float32),
        scratch_types=[
            pltpu.VMEM((_CH,), jnp.int32),
            pltpu.VMEM((_CH, 2 * _C), jnp.float32),
            pltpu.SemaphoreType.DMA,
        ],
    )
    def k(table_hbm, idx_hbm, out_hbm, idx_v, rows_v, sem):
        wid = lax.axis_index("s") * info.num_cores + lax.axis_index("c")
        base0 = wid * per_w

        @pl.loop(0, n_ch)
        def _(j):
            base = base0 + j * _CH
            pltpu.sync_copy(idx_hbm.at[pl.ds(base, _CH)], idx_v)
            pltpu.async_copy(table_hbm.at[idx_v], rows_v, sem).wait()
            pltpu.sync_copy(rows_v, out_hbm.at[pl.ds(base, _CH)])

    return k(table, idx_flat)


def _mlp_body(a_ref, g_ref, w2t_ref, b2_ref, gamma_ref, beta_ref, o_ref):
    a = a_ref[...]
    acc = jnp.zeros((_R2, _C), jnp.float32)
    for t in range(_K):
        h1 = _leaky(a + g_ref[t][:, :_C])
        z = lax.dot_general(h1, w2t_ref[...], (((1,), (0,)), ((), ())),
                            preferred_element_type=jnp.float32,
                            precision=_DEFAULT) + b2_ref[...]
        acc = acc + _leaky(z)
    agg = acc * (1.0 / _K)
    mu = jnp.mean(agg, axis=1, keepdims=True)
    d = agg - mu
    var = jnp.mean(d * d, axis=1, keepdims=True)
    o_ref[...] = (d / jnp.sqrt(var + 1e-5)) * gamma_ref[...] + beta_ref[...]


def _mlp(a2, g, w2t, b2_2d, gamma_2d, beta_2d):
    npts = a2.shape[0]
    return pl.pallas_call(
        _mlp_body,
        grid=(npts // _R2,),
        in_specs=[
            pl.BlockSpec((_R2, _C), lambda i: (i, 0)),
            pl.BlockSpec((_K, _R2, 2 * _C), lambda i: (0, i, 0)),
            pl.BlockSpec((_C, _C), lambda i: (0, 0)),
            pl.BlockSpec((1, _C), lambda i: (0, 0)),
            pl.BlockSpec((1, _C), lambda i: (0, 0)),
            pl.BlockSpec((1, _C), lambda i: (0, 0)),
        ],
        out_specs=pl.BlockSpec((_R2, _C), lambda i: (i, 0)),
        out_shape=jax.ShapeDtypeStruct((npts, _C), jnp.float32),
    )(a2, g, w2t, b2_2d, gamma_2d, beta_2d)


def kernel(nodes, w1, b1, w2, b2, gamma, beta):
    b_, p_, c_ = nodes.shape
    w1t = w1.T
    wa = w1t[:c_]
    wb = w1t[c_:]
    wd = wa - wb
    b1_2d = b1.reshape(1, c_)
    w2t = w2.T
    b2_2d = b2.reshape(1, c_)
    gamma_2d = gamma.reshape(1, c_)
    beta_2d = beta.reshape(1, c_)
    outs = []
    for b in range(b_):
        knn, a, nb = _sim_topk(nodes[b], wd, wb, b1_2d)
        idx_flat = knn.T.reshape(-1)
        g = _sc_gather(nb, idx_flat)
        g = g.reshape(_K, p_, 2 * c_)
        outs.append(_mlp(a, g, w2t, b2_2d, gamma_2d, beta_2d))
    return jnp.stack(outs, axis=0)
